# Initial kernel scaffold; baseline (speedup 1.0000x reference)
import jax
import jax.numpy as jnp
from jax.experimental import pallas as pl


def kernel(score):
    raise NotImplementedError("write your pallas kernel here")


# speedup vs baseline: 16.1612x; 16.1612x over previous
import jax
import jax.numpy as jnp
from jax.experimental import pallas as pl

_B, _F, _P, _N = 64, 16, 576, 24


def _body(score_ref, out_ref):
    patch_iota = jax.lax.broadcasted_iota(jnp.int32, (_B, _P), 1)
    col = patch_iota % _N
    b = jnp.where((patch_iota >= 4 * _N) & (col >= 2) & (col < _N - 2),
                  1.0, 0.0).astype(jnp.float32)
    out_ref[:, 0:1] = jnp.ones((_B, 1), jnp.float32)
    not_col0 = (col != 0).astype(jnp.float32)
    not_colL = (col != _N - 1).astype(jnp.float32)
    zc24 = jnp.zeros((_B, _N), jnp.float32)
    zc1 = jnp.zeros((_B, 1), jnp.float32)
    for i in range(_F):
        a = score_ref[:, i, :]
        m = jnp.max(a, axis=1, keepdims=True)
        idxv = jnp.where(a == m, patch_iota, _P)
        first = jnp.min(idxv, axis=1, keepdims=True)
        onehot = patch_iota == first
        up = jnp.concatenate([zc24, b[:, : _P - _N]], axis=1)
        down = jnp.concatenate([b[:, _N:], zc24], axis=1)
        left = jnp.concatenate([zc1, b[:, : _P - 1]], axis=1) * not_col0
        right = jnp.concatenate([b[:, 1:], zc1], axis=1) * not_colL
        nm = up + down + left + right
        newbit = jnp.where(onehot & (nm > 0.0), 1.0, 0.0)
        b = jnp.maximum(b, newbit)
        out_ref[:, 1 + _P * i : 1 + _P * (i + 1)] = b


@jax.jit
def kernel(score):
    return pl.pallas_call(
        _body,
        out_shape=jax.ShapeDtypeStruct((_B, 1 + _F * _P), jnp.float32),
    )(score)


# speedup vs baseline: 22.6551x; 1.4018x over previous
import jax
import jax.numpy as jnp
from jax.experimental import pallas as pl

_B, _F, _P, _N = 64, 16, 576, 24


def _body(score_ref, out_ref):
    iota2 = jax.lax.broadcasted_iota(jnp.int32, (_B * _F, _P), 1)
    s = score_ref[...].reshape(_B * _F, _P)
    m = jnp.max(s, axis=1, keepdims=True)
    firsts = jnp.min(jnp.where(s == m, iota2, _P), axis=1).reshape(_B, _F)

    patch_iota = jax.lax.broadcasted_iota(jnp.int32, (_B, _P), 1)
    col = patch_iota % _N
    not_col0 = col != 0
    not_colL = col != _N - 1
    b = jnp.where((patch_iota >= 4 * _N) & (col >= 2) & (col < _N - 2),
                  1.0, 0.0).astype(jnp.float32)
    out_ref[:, 0:1] = jnp.ones((_B, 1), jnp.float32)
    for i in range(_F):
        first = firsts[:, i].reshape(_B, 1)
        diff = patch_iota - first
        adj = ((diff == -_N) | (diff == _N)
               | ((diff == -1) & not_colL) | ((diff == 1) & not_col0))
        hit = jnp.max(jnp.where(adj, b, 0.0), axis=1, keepdims=True)
        newbit = jnp.where((diff == 0) & (hit > 0.0), 1.0, 0.0)
        b = jnp.maximum(b, newbit)
        out_ref[:, 1 + _P * i : 1 + _P * (i + 1)] = b


@jax.jit
def kernel(score):
    return pl.pallas_call(
        _body,
        out_shape=jax.ShapeDtypeStruct((_B, 1 + _F * _P), jnp.float32),
    )(score)
